# baseline (speedup 1.0000x reference)
import functools

import jax
import jax.numpy as jnp
from jax import lax
from jax.experimental import pallas as pl
from jax.experimental.pallas import tpu as pltpu
from jax.experimental.pallas import tpu_sc as plsc

N_EXP = 64
N_GRP = 8
EPG = 8
TOPK_G = 3
TOPK = 8
LANES = 16
INT_MIN = -2147483648

_SORT8 = ((0, 1), (2, 3), (4, 5), (6, 7),
          (0, 2), (1, 3), (4, 6), (5, 7),
          (1, 2), (5, 6),
          (0, 4), (1, 5), (2, 6), (3, 7),
          (2, 4), (3, 5),
          (1, 2), (3, 4), (5, 6))


def _sort8_desc(v):
    v = list(v)
    for i, j in _SORT8:
        hi = jnp.maximum(v[i], v[j])
        lo = jnp.minimum(v[i], v[j])
        v[i], v[j] = hi, lo
    return v


def _merge_top8(a, b):
    c = [jnp.maximum(a[i], b[7 - i]) for i in range(8)]
    for d in (4, 2, 1):
        nxt = list(c)
        for i in range(8):
            if i % (2 * d) < d:
                nxt[i] = jnp.maximum(c[i], c[i + d])
                nxt[i + d] = jnp.minimum(c[i], c[i + d])
        c = nxt
    return c


def _make_logits_kernel(wpb, tpw):
    def _logits_kernel(w_ref, h_ref, lt_ref, gm_ref):
        logits = jax.lax.dot_general(
            w_ref[...], h_ref[...],
            (((1,), (1,)), ((), ())),
            preferred_element_type=jnp.float32,
        )
        gs = jnp.concatenate(
            [jnp.max(logits[g * EPG:(g + 1) * EPG], axis=0, keepdims=True)
             for g in range(N_GRP)], axis=0)
        for k in range(wpb):
            lt_ref[k] = logits[:, k * tpw:(k + 1) * tpw]
            gm_ref[k] = gs[:, k * tpw:(k + 1) * tpw]
    return _logits_kernel


def _tree_reduce(vals, op):
    while len(vals) > 1:
        nxt = [op(vals[i], vals[i + 1]) for i in range(0, len(vals) - 1, 2)]
        if len(vals) % 2:
            nxt.append(vals[-1])
        vals = nxt
    return vals[0]


def _sortable(v):
    b = plsc.bitcast(v, jnp.int32)
    return jnp.where(b < 0, b ^ 0x7FFFFFFF, b)


def _make_sc_router(nw, nc, tpw):
    nchunk = tpw // LANES
    mesh = plsc.VectorSubcoreMesh(core_axis_name="c", subcore_axis_name="s")

    @functools.partial(
        pl.kernel,
        out_type=jax.ShapeDtypeStruct((nw * tpw * TOPK,), jnp.float32),
        mesh=mesh,
        scratch_types=[
            pltpu.VMEM((N_EXP, tpw), jnp.float32),
            pltpu.VMEM((N_GRP, tpw), jnp.float32),
            pltpu.VMEM((tpw * TOPK,), jnp.float32),
        ],
        compiler_params=pltpu.CompilerParams(needs_layout_passes=False),
    )
    def router(lt_hbm, gm_hbm, out_hbm, lt_v, gm_v, out_v):
        wid = lax.axis_index("s") * nc + lax.axis_index("c")
        pltpu.sync_copy(lt_hbm.at[wid], lt_v)
        pltpu.sync_copy(gm_hbm.at[wid], gm_v)
        iota = lax.iota(jnp.int32, LANES)

        @plsc.parallel_loop(0, nchunk, unroll=2)
        def chunk_body(c):
            off = c * LANES
            pos = off + iota

            gk = [(_sortable(gm_v[g, pl.ds(off, LANES)]) & -8) | (7 - g)
                  for g in range(N_GRP)]

            sel_g = []
            for _ in range(TOPK_G):
                m = _tree_reduce(list(gk), jnp.maximum)
                g_r = 7 - (m & 7)
                sel_g.append(g_r)
                gk = [jnp.where(g_r == g, INT_MIN, gk[g])
                      for g in range(N_GRP)]

            groups = []
            for r in range(TOPK_G):
                ebase = sel_g[r] * EPG
                groups.append([
                    plsc.load_gather(lt_v, [ebase + j, pos])
                    for j in range(EPG)])

            g0 = _sort8_desc(groups[0])
            g1 = _sort8_desc(groups[1])
            g2 = _sort8_desc(groups[2])
            vals = _merge_top8(_merge_top8(g0, g1), g2)

            ex = [jnp.exp(v - vals[0]) for v in vals]
            s = _tree_reduce(list(ex), jnp.add)
            rcp = 1.0 / s
            pos8 = pos * TOPK
            for i in range(TOPK):
                plsc.store_scatter(out_v, [pos8 + i], ex[i] * rcp)

        del chunk_body
        pltpu.sync_copy(
            out_v, out_hbm.at[pl.ds(wid * tpw * TOPK, tpw * TOPK)])

    return router


def kernel(hidden_states, kernel):
    gate_w = kernel
    S, H = hidden_states.shape
    T = 2048
    P = 1
    NW = 32
    SC_TOK = S // P
    tpw = SC_TOK // NW
    wpb = T // tpw
    router = _make_sc_router(NW, 2, tpw)
    logits_fn = _make_logits_kernel(wpb, tpw)
    outs = []
    for p in range(P):
        off = p * (SC_TOK // T)
        mm = pl.pallas_call(
            logits_fn,
            grid=(SC_TOK // T,),
            in_specs=[
                pl.BlockSpec((N_EXP, H), lambda i: (0, 0)),
                pl.BlockSpec((T, H), lambda i, o=off: (i + o, 0)),
            ],
            out_specs=(
                pl.BlockSpec((wpb, N_EXP, tpw), lambda i: (i, 0, 0)),
                pl.BlockSpec((wpb, N_GRP, tpw), lambda i: (i, 0, 0)),
            ),
            out_shape=(
                jax.ShapeDtypeStruct((NW, N_EXP, tpw), jnp.float32),
                jax.ShapeDtypeStruct((NW, N_GRP, tpw), jnp.float32),
            ),
        )
        lt, gm = mm(gate_w, hidden_states)
        outs.append(jnp.reshape(router(lt, gm), (SC_TOK, TOPK)))
    return outs[0] if P == 1 else jnp.concatenate(outs, axis=0)


# speedup vs baseline: 1.0466x; 1.0466x over previous
import functools

import jax
import jax.numpy as jnp
from jax import lax
from jax.experimental import pallas as pl
from jax.experimental.pallas import tpu as pltpu
from jax.experimental.pallas import tpu_sc as plsc

N_EXP = 64
N_GRP = 8
EPG = 8
TOPK_G = 3
TOPK = 8
LANES = 16
INT_MIN = -2147483648

_SORT8 = ((0, 1), (2, 3), (4, 5), (6, 7),
          (0, 2), (1, 3), (4, 6), (5, 7),
          (1, 2), (5, 6),
          (0, 4), (1, 5), (2, 6), (3, 7),
          (2, 4), (3, 5),
          (1, 2), (3, 4), (5, 6))


def _sort8_desc(v):
    v = list(v)
    for i, j in _SORT8:
        hi = jnp.maximum(v[i], v[j])
        lo = jnp.minimum(v[i], v[j])
        v[i], v[j] = hi, lo
    return v


def _merge_top8(a, b):
    c = [jnp.maximum(a[i], b[7 - i]) for i in range(8)]
    for d in (4, 2, 1):
        nxt = list(c)
        for i in range(8):
            if i % (2 * d) < d:
                nxt[i] = jnp.maximum(c[i], c[i + d])
                nxt[i + d] = jnp.minimum(c[i], c[i + d])
        c = nxt
    return c


def _make_logits_kernel(wpb, tpw):
    def _logits_kernel(w_ref, h_ref, lt_ref, gm_ref):
        logits = jax.lax.dot_general(
            w_ref[...], h_ref[...],
            (((1,), (1,)), ((), ())),
            preferred_element_type=jnp.float32,
        )
        gs = jnp.concatenate(
            [jnp.max(logits[g * EPG:(g + 1) * EPG], axis=0, keepdims=True)
             for g in range(N_GRP)], axis=0)
        for k in range(wpb):
            lt_ref[k] = logits[:, k * tpw:(k + 1) * tpw]
            gm_ref[k] = gs[:, k * tpw:(k + 1) * tpw]
    return _logits_kernel


def _tree_reduce(vals, op):
    while len(vals) > 1:
        nxt = [op(vals[i], vals[i + 1]) for i in range(0, len(vals) - 1, 2)]
        if len(vals) % 2:
            nxt.append(vals[-1])
        vals = nxt
    return vals[0]


def _sortable(v):
    b = plsc.bitcast(v, jnp.int32)
    return jnp.where(b < 0, b ^ 0x7FFFFFFF, b)


def _make_sc_router(nw, nc, tpw):
    nchunk = tpw // LANES
    mesh = plsc.VectorSubcoreMesh(core_axis_name="c", subcore_axis_name="s")

    @functools.partial(
        pl.kernel,
        out_type=jax.ShapeDtypeStruct((nw * tpw, TOPK), jnp.float32),
        mesh=mesh,
        scratch_types=[
            pltpu.VMEM((N_EXP, tpw), jnp.float32),
            pltpu.VMEM((N_GRP, tpw), jnp.float32),
            pltpu.VMEM((tpw, TOPK), jnp.float32),
        ],
        compiler_params=pltpu.CompilerParams(needs_layout_passes=False),
    )
    def router(lt_hbm, gm_hbm, out_hbm, lt_v, gm_v, out_v):
        wid = lax.axis_index("s") * nc + lax.axis_index("c")
        pltpu.sync_copy(lt_hbm.at[wid], lt_v)
        pltpu.sync_copy(gm_hbm.at[wid], gm_v)
        iota = lax.iota(jnp.int32, LANES)

        @plsc.parallel_loop(0, nchunk, unroll=4)
        def chunk_body(c):
            off = c * LANES
            pos = off + iota

            gk = [(_sortable(gm_v[g, pl.ds(off, LANES)]) & -8) | (7 - g)
                  for g in range(N_GRP)]

            sel_g = []
            for _ in range(TOPK_G):
                m = _tree_reduce(list(gk), jnp.maximum)
                g_r = 7 - (m & 7)
                sel_g.append(g_r)
                gk = [jnp.where(g_r == g, INT_MIN, gk[g])
                      for g in range(N_GRP)]

            groups = []
            for r in range(TOPK_G):
                ebase = sel_g[r] * EPG
                groups.append([
                    plsc.load_gather(lt_v, [ebase + j, pos])
                    for j in range(EPG)])

            g0 = _sort8_desc(groups[0])
            g1 = _sort8_desc(groups[1])
            g2 = _sort8_desc(groups[2])
            vals = _merge_top8(_merge_top8(g0, g1), g2)

            ex = [jnp.exp(v - vals[0]) for v in vals]
            s = _tree_reduce(list(ex), jnp.add)
            rcp = 1.0 / s
            for i in range(TOPK):
                plsc.store_scatter(
                    out_v, [pos, jnp.full((LANES,), i, jnp.int32)],
                    ex[i] * rcp)

        del chunk_body
        pltpu.sync_copy(out_v, out_hbm.at[pl.ds(wid * tpw, tpw), :])

    return router


def kernel(hidden_states, kernel):
    gate_w = kernel
    S, H = hidden_states.shape
    T = 2048
    P = 1
    NW = 32
    SC_TOK = S // P
    tpw = SC_TOK // NW
    wpb = T // tpw
    router = _make_sc_router(NW, 2, tpw)
    logits_fn = _make_logits_kernel(wpb, tpw)
    outs = []
    for p in range(P):
        off = p * (SC_TOK // T)
        mm = pl.pallas_call(
            logits_fn,
            grid=(SC_TOK // T,),
            in_specs=[
                pl.BlockSpec((N_EXP, H), lambda i: (0, 0)),
                pl.BlockSpec((T, H), lambda i, o=off: (i + o, 0)),
            ],
            out_specs=(
                pl.BlockSpec((wpb, N_EXP, tpw), lambda i: (i, 0, 0)),
                pl.BlockSpec((wpb, N_GRP, tpw), lambda i: (i, 0, 0)),
            ),
            out_shape=(
                jax.ShapeDtypeStruct((NW, N_EXP, tpw), jnp.float32),
                jax.ShapeDtypeStruct((NW, N_GRP, tpw), jnp.float32),
            ),
        )
        lt, gm = mm(gate_w, hidden_states)
        outs.append(router(lt, gm))
    return outs[0] if P == 1 else jnp.concatenate(outs, axis=0)


# speedup vs baseline: 1.2065x; 1.1528x over previous
import functools

import jax
import jax.numpy as jnp
from jax import lax
from jax.experimental import pallas as pl
from jax.experimental.pallas import tpu as pltpu
from jax.experimental.pallas import tpu_sc as plsc

N_EXP = 64
N_GRP = 8
EPG = 8
TOPK_G = 3
TOPK = 8
LANES = 16
INT_MIN = -2147483648

_SORT8 = ((0, 1), (2, 3), (4, 5), (6, 7),
          (0, 2), (1, 3), (4, 6), (5, 7),
          (1, 2), (5, 6),
          (0, 4), (1, 5), (2, 6), (3, 7),
          (2, 4), (3, 5),
          (1, 2), (3, 4), (5, 6))


def _sort8_desc(v):
    v = list(v)
    for i, j in _SORT8:
        hi = jnp.maximum(v[i], v[j])
        lo = jnp.minimum(v[i], v[j])
        v[i], v[j] = hi, lo
    return v


def _merge_top8(a, b):
    c = [jnp.maximum(a[i], b[7 - i]) for i in range(8)]
    for d in (4, 2, 1):
        nxt = list(c)
        for i in range(8):
            if i % (2 * d) < d:
                nxt[i] = jnp.maximum(c[i], c[i + d])
                nxt[i + d] = jnp.minimum(c[i], c[i + d])
        c = nxt
    return c


def _make_logits_kernel(wpb, tpw):
    def _logits_kernel(w_ref, h_ref, lt_ref, gm_ref):
        logits = jax.lax.dot_general(
            w_ref[...], h_ref[...],
            (((1,), (1,)), ((), ())),
            preferred_element_type=jnp.float32,
        )
        gs = jnp.concatenate(
            [jnp.max(logits[g * EPG:(g + 1) * EPG], axis=0, keepdims=True)
             for g in range(N_GRP)], axis=0)
        for k in range(wpb):
            lt_ref[k] = logits[:, k * tpw:(k + 1) * tpw]
            gm_ref[k] = gs[:, k * tpw:(k + 1) * tpw]
    return _logits_kernel


def _tree_reduce(vals, op):
    while len(vals) > 1:
        nxt = [op(vals[i], vals[i + 1]) for i in range(0, len(vals) - 1, 2)]
        if len(vals) % 2:
            nxt.append(vals[-1])
        vals = nxt
    return vals[0]


def _sortable(v):
    b = plsc.bitcast(v, jnp.int32)
    return jnp.where(b < 0, b ^ 0x7FFFFFFF, b)


def _make_sc_router(nw, nc, tpw):
    nchunk = tpw // LANES
    mesh = plsc.VectorSubcoreMesh(core_axis_name="c", subcore_axis_name="s")

    @functools.partial(
        pl.kernel,
        out_type=jax.ShapeDtypeStruct((TOPK, nw * tpw), jnp.float32),
        mesh=mesh,
        scratch_types=[
            pltpu.VMEM((N_EXP, tpw), jnp.float32),
            pltpu.VMEM((N_GRP, tpw), jnp.float32),
            pltpu.VMEM((TOPK, tpw), jnp.float32),
        ],
        compiler_params=pltpu.CompilerParams(needs_layout_passes=False),
    )
    def router(lt_hbm, gm_hbm, out_hbm, lt_v, gm_v, out_v):
        wid = lax.axis_index("s") * nc + lax.axis_index("c")
        pltpu.sync_copy(lt_hbm.at[wid], lt_v)
        pltpu.sync_copy(gm_hbm.at[wid], gm_v)
        iota = lax.iota(jnp.int32, LANES)

        @plsc.parallel_loop(0, nchunk, unroll=4)
        def chunk_body(c):
            off = c * LANES
            pos = off + iota

            gk = [(_sortable(gm_v[g, pl.ds(off, LANES)]) & -8) | (7 - g)
                  for g in range(N_GRP)]

            sel_g = []
            for _ in range(TOPK_G):
                m = _tree_reduce(list(gk), jnp.maximum)
                g_r = 7 - (m & 7)
                sel_g.append(g_r)
                gk = [jnp.where(g_r == g, INT_MIN, gk[g])
                      for g in range(N_GRP)]

            groups = []
            for r in range(TOPK_G):
                ebase = sel_g[r] * EPG
                groups.append([
                    plsc.load_gather(lt_v, [ebase + j, pos])
                    for j in range(EPG)])

            g0 = _sort8_desc(groups[0])
            g1 = _sort8_desc(groups[1])
            g2 = _sort8_desc(groups[2])
            vals = _merge_top8(_merge_top8(g0, g1), g2)

            ex = [jnp.exp(v - vals[0]) for v in vals]
            s = _tree_reduce(list(ex), jnp.add)
            rcp = 1.0 / s
            for i in range(TOPK):
                out_v[i, pl.ds(off, LANES)] = ex[i] * rcp

        del chunk_body
        pltpu.sync_copy(out_v, out_hbm.at[:, pl.ds(wid * tpw, tpw)])

    return router


def kernel(hidden_states, kernel):
    gate_w = kernel
    S, H = hidden_states.shape
    T = 2048
    P = 1
    NW = 32
    SC_TOK = S // P
    tpw = SC_TOK // NW
    wpb = T // tpw
    router = _make_sc_router(NW, 2, tpw)
    logits_fn = _make_logits_kernel(wpb, tpw)
    outs = []
    for p in range(P):
        off = p * (SC_TOK // T)
        mm = pl.pallas_call(
            logits_fn,
            grid=(SC_TOK // T,),
            in_specs=[
                pl.BlockSpec((N_EXP, H), lambda i: (0, 0)),
                pl.BlockSpec((T, H), lambda i, o=off: (i + o, 0)),
            ],
            out_specs=(
                pl.BlockSpec((wpb, N_EXP, tpw), lambda i: (i, 0, 0)),
                pl.BlockSpec((wpb, N_GRP, tpw), lambda i: (i, 0, 0)),
            ),
            out_shape=(
                jax.ShapeDtypeStruct((NW, N_EXP, tpw), jnp.float32),
                jax.ShapeDtypeStruct((NW, N_GRP, tpw), jnp.float32),
            ),
        )
        lt, gm = mm(gate_w, hidden_states)
        outs.append(router(lt, gm).T)
    return outs[0] if P == 1 else jnp.concatenate(outs, axis=0)
